# Initial kernel scaffold; baseline (speedup 1.0000x reference)
import jax
import jax.numpy as jnp
from jax.experimental import pallas as pl


def kernel(points, colors):
    raise NotImplementedError("write your pallas kernel here")


# speedup vs baseline: 1.8566x; 1.8566x over previous
import jax
import jax.numpy as jnp
from jax import lax
from jax.experimental import pallas as pl
from jax.experimental.pallas import tpu as pltpu
from jax.experimental.pallas import tpu_sc as plsc

_RES = 256
_L = 16
_NC, _NS = 2, 16
_NW = _NC * _NS
_P = 4096


def _body(points_hbm, colors_hbm, out_hbm, in_v, tab_v, out_v):
    wid = lax.axis_index("s") * _NC + lax.axis_index("c")
    per_tile = points_hbm.shape[0] // _NW
    chunks = per_tile // _P

    pltpu.sync_copy(colors_hbm, tab_v)
    iota = lax.iota(jnp.int32, _L)

    def chunk_body(t, carry):
        base = wid * per_tile + t * _P
        pltpu.sync_copy(points_hbm.at[pl.ds(base, _P)], in_v)

        def step(j, c2):
            rows = j * _L + iota
            d = plsc.load_gather(in_v, [rows, jnp.full((_L,), 3, jnp.int32)])
            idx = jnp.clip(d.astype(jnp.int32), 0, _RES - 1)
            for c in range(3):
                v = plsc.load_gather(in_v, [rows, jnp.full((_L,), c, jnp.int32)])
                plsc.store_scatter(out_v, [rows, jnp.full((_L,), c, jnp.int32)], v)
            for c in range(3):
                col = plsc.load_gather(tab_v, [idx, jnp.full((_L,), c, jnp.int32)])
                col = jnp.clip(col, 0.0, 1.0)
                plsc.store_scatter(
                    out_v, [rows, jnp.full((_L,), 3 + c, jnp.int32)], col)
            return c2

        lax.fori_loop(0, _P // _L, step, 0)
        pltpu.sync_copy(out_v, out_hbm.at[pl.ds(base, _P)])
        return carry

    lax.fori_loop(0, chunks, chunk_body, 0)


def kernel(points, colors):
    n = points.shape[0]
    f = pl.kernel(
        _body,
        out_type=jax.ShapeDtypeStruct((n, 6), jnp.float32),
        mesh=plsc.VectorSubcoreMesh(
            core_axis_name="c", subcore_axis_name="s",
            num_cores=_NC, num_subcores=_NS),
        scratch_types=[
            pltpu.VMEM((_P, 4), jnp.float32),
            pltpu.VMEM((_RES, 3), jnp.float32),
            pltpu.VMEM((_P, 6), jnp.float32),
        ],
        compiler_params=pltpu.CompilerParams(
            needs_layout_passes=False, use_tc_tiling_on_sc=False),
    )
    return f(points, colors)


# speedup vs baseline: 1.9382x; 1.0440x over previous
import jax
import jax.numpy as jnp
from jax import lax
from jax.experimental import pallas as pl
from jax.experimental.pallas import tpu as pltpu
from jax.experimental.pallas import tpu_sc as plsc

_RES = 256
_L = 16
_NC, _NS = 2, 16
_NW = _NC * _NS
_P = 4096
_NBUF = 2


def _compute_chunk(in_v, tab_v, out_v):
    iota = lax.iota(jnp.int32, _L)

    @plsc.parallel_loop(0, _P // _L, unroll=8)
    def _(j):
        rows = j * _L + iota
        pos4 = rows * 4
        pos6 = rows * 6
        d = plsc.load_gather(in_v, [pos4 + 3])
        idx3 = jnp.clip(d.astype(jnp.int32), 0, _RES - 1) * 3
        for c in range(3):
            v = plsc.load_gather(in_v, [pos4 + c])
            plsc.store_scatter(out_v, [pos6 + c], v)
            col = plsc.load_gather(tab_v, [idx3 + c])
            col = jnp.clip(col, 0.0, 1.0)
            plsc.store_scatter(out_v, [pos6 + (3 + c)], col)


def _body(points_hbm, colors_hbm, out_hbm, tab_v,
          in_v0, in_v1, out_v0, out_v1, sin0, sin1, sout0, sout1):
    ins, outs = [in_v0, in_v1], [out_v0, out_v1]
    sins, souts = [sin0, sin1], [sout0, sout1]
    wid = lax.axis_index("s") * _NC + lax.axis_index("c")
    per_tile = points_hbm.shape[0] // (4 * _NW)
    chunks = per_tile // _P
    tile_base = wid * per_tile

    pltpu.sync_copy(colors_hbm, tab_v)
    for b in range(_NBUF):
        pltpu.async_copy(
            points_hbm.at[pl.ds((tile_base + b * _P) * 4, _P * 4)],
            ins[b], sins[b])

    def outer(g, carry):
        for b in range(_NBUF):
            t = g * _NBUF + b
            base = tile_base + t * _P
            pltpu.make_async_copy(
                points_hbm.at[pl.ds(base * 4, _P * 4)], ins[b], sins[b]).wait()

            @pl.when(t >= _NBUF)
            def _():
                prev = tile_base + (t - _NBUF) * _P
                pltpu.make_async_copy(
                    outs[b], out_hbm.at[pl.ds(prev * 6, _P * 6)],
                    souts[b]).wait()

            _compute_chunk(ins[b], tab_v, outs[b])
            pltpu.async_copy(
                outs[b], out_hbm.at[pl.ds(base * 6, _P * 6)], souts[b])

            @pl.when(t + _NBUF < chunks)
            def _():
                nxt = tile_base + (t + _NBUF) * _P
                pltpu.async_copy(
                    points_hbm.at[pl.ds(nxt * 4, _P * 4)], ins[b], sins[b])
        return carry

    lax.fori_loop(0, chunks // _NBUF, outer, 0)
    for b in range(_NBUF):
        base = tile_base + (chunks - _NBUF + b) * _P
        pltpu.make_async_copy(
            outs[b], out_hbm.at[pl.ds(base * 6, _P * 6)], souts[b]).wait()


def kernel(points, colors):
    n = points.shape[0]
    f = pl.kernel(
        _body,
        out_type=jax.ShapeDtypeStruct((n * 6,), jnp.float32),
        mesh=plsc.VectorSubcoreMesh(
            core_axis_name="c", subcore_axis_name="s",
            num_cores=_NC, num_subcores=_NS),
        scratch_types=[
            pltpu.VMEM((_RES * 3,), jnp.float32),
            pltpu.VMEM((_P * 4,), jnp.float32),
            pltpu.VMEM((_P * 4,), jnp.float32),
            pltpu.VMEM((_P * 6,), jnp.float32),
            pltpu.VMEM((_P * 6,), jnp.float32),
            pltpu.SemaphoreType.DMA,
            pltpu.SemaphoreType.DMA,
            pltpu.SemaphoreType.DMA,
            pltpu.SemaphoreType.DMA,
        ],
        compiler_params=pltpu.CompilerParams(
            needs_layout_passes=False, use_tc_tiling_on_sc=False),
    )
    out_flat = f(points.reshape(-1), colors.reshape(-1))
    return out_flat.reshape(n, 6)


# speedup vs baseline: 90.4427x; 46.6627x over previous
import jax
import jax.numpy as jnp
from jax import lax
from jax.experimental import pallas as pl
from jax.experimental.pallas import tpu as pltpu
from jax.experimental.pallas import tpu_sc as plsc

_RES = 256
_L = 16
_NC, _NS = 2, 16
_NW = _NC * _NS
_B = 128
_P = 4096
_NBUF = 2


def _compute_chunk(in_v, tab_v, out_v):
    @plsc.parallel_loop(0, _P // _B, unroll=2)
    def _(gi):
        ib = gi * (4 * _B)
        ob = gi * (8 * _B)
        for k in range(_B // _L):
            o = k * _L
            w = in_v[pl.ds(ib + 3 * _B + o, _L)]
            idx3 = jnp.clip(w.astype(jnp.int32), 0, _RES - 1) * 3
            for c in range(3):
                out_v[pl.ds(ob + c * _B + o, _L)] = \
                    in_v[pl.ds(ib + c * _B + o, _L)]
                col = plsc.load_gather(tab_v, [idx3 + c])
                out_v[pl.ds(ob + (3 + c) * _B + o, _L)] = \
                    jnp.clip(col, 0.0, 1.0)


def _body(points_hbm, colors_hbm, out_hbm, tab_v,
          in_v0, in_v1, out_v0, out_v1, sin0, sin1, sout0, sout1):
    ins, outs = [in_v0, in_v1], [out_v0, out_v1]
    sins, souts = [sin0, sin1], [sout0, sout1]
    wid = lax.axis_index("s") * _NC + lax.axis_index("c")
    per_tile = points_hbm.shape[0] // (4 * _NW)
    chunks = per_tile // _P
    tile_base = wid * per_tile

    pltpu.sync_copy(colors_hbm, tab_v)
    for b in range(_NBUF):
        pltpu.async_copy(
            points_hbm.at[pl.ds((tile_base + b * _P) * 4, _P * 4)],
            ins[b], sins[b])

    def outer(g, carry):
        for b in range(_NBUF):
            t = g * _NBUF + b
            base = tile_base + t * _P
            pltpu.make_async_copy(
                points_hbm.at[pl.ds(base * 4, _P * 4)], ins[b], sins[b]).wait()

            @pl.when(t >= _NBUF)
            def _():
                prev = tile_base + (t - _NBUF) * _P
                pltpu.make_async_copy(
                    outs[b], out_hbm.at[pl.ds(prev * 8, _P * 8)],
                    souts[b]).wait()

            _compute_chunk(ins[b], tab_v, outs[b])
            pltpu.async_copy(
                outs[b], out_hbm.at[pl.ds(base * 8, _P * 8)], souts[b])

            @pl.when(t + _NBUF < chunks)
            def _():
                nxt = tile_base + (t + _NBUF) * _P
                pltpu.async_copy(
                    points_hbm.at[pl.ds(nxt * 4, _P * 4)], ins[b], sins[b])
        return carry

    lax.fori_loop(0, chunks // _NBUF, outer, 0)
    for b in range(_NBUF):
        base = tile_base + (chunks - _NBUF + b) * _P
        pltpu.make_async_copy(
            outs[b], out_hbm.at[pl.ds(base * 8, _P * 8)], souts[b]).wait()


def kernel(points, colors):
    n = points.shape[0]
    g = n // _B
    f = pl.kernel(
        _body,
        out_type=jax.ShapeDtypeStruct((n * 8,), jnp.float32),
        mesh=plsc.VectorSubcoreMesh(
            core_axis_name="c", subcore_axis_name="s",
            num_cores=_NC, num_subcores=_NS),
        scratch_types=[
            pltpu.VMEM((_RES * 3,), jnp.float32),
            pltpu.VMEM((_P * 4,), jnp.float32),
            pltpu.VMEM((_P * 4,), jnp.float32),
            pltpu.VMEM((_P * 8,), jnp.float32),
            pltpu.VMEM((_P * 8,), jnp.float32),
            pltpu.SemaphoreType.DMA,
            pltpu.SemaphoreType.DMA,
            pltpu.SemaphoreType.DMA,
            pltpu.SemaphoreType.DMA,
        ],
        compiler_params=pltpu.CompilerParams(
            needs_layout_passes=False, use_tc_tiling_on_sc=False),
    )
    pts_soa = points.reshape(g, _B, 4).transpose(0, 2, 1).reshape(n * 4)
    out8 = f(pts_soa, colors.reshape(-1))
    return out8.reshape(g, 8, _B)[:, :6, :].transpose(0, 2, 1).reshape(n, 6)
